# Initial kernel scaffold; baseline (speedup 1.0000x reference)
import functools

import jax
import jax.numpy as jnp
from jax import lax
from jax.experimental import pallas as pl
from jax.experimental.pallas import tpu as pltpu
from jax.experimental.pallas import tpu_sc as plsc

ROWS = 128
N = 8192
TOPK = 64
L = 16
NCHUNK = N // L
NCORES = 2
NSUB = 16
NW = NCORES * NSUB
RPW = ROWS // NW
HWORDS = L * 256

MINT = jnp.int32(-2147483648)
M7F = jnp.int32(0x7FFFFFFF)


def _bin_scan(hist_v, tot_v, rank):
    lane = lax.iota(jnp.int32, L)

    def tot_body(g, _):
        def lane_body(l, acc):
            return acc + hist_v[pl.ds(l * 256 + g * L, L)]

        tot_v[pl.ds(g * L, L)] = lax.fori_loop(
            0, L, lane_body, jnp.zeros((L,), jnp.int32)
        )
        return 0

    lax.fori_loop(0, 16, tot_body, 0)

    def sweep(g9, carry):
        carry_cnt, pc_tot = carry
        g = 15 - g9
        v = tot_v[pl.ds(g * L, L)]
        sfx = lax.rev(plsc.cumsum(lax.rev(v, (0,))), (0,))
        m = (sfx + carry_cnt) >= rank
        pc_tot = pc_tot + plsc.all_reduce_population_count(m)
        return carry_cnt + jnp.sum(v), pc_tot

    _, pc_tot = lax.fori_loop(
        0, 16, sweep, (jnp.int32(0), jnp.zeros((L,), jnp.int32))
    )
    bstar = pc_tot - 1

    def above(g, acc):
        v = tot_v[pl.ds(g * L, L)]
        binidx = g * L + lane
        return acc + jnp.sum(jnp.where(binidx > bstar, v, 0))

    cnt_above = lax.fori_loop(0, 16, above, jnp.int32(0))
    return bstar, rank - cnt_above


def _body(in_hbm, out_hbm, x_v, sk_v, ci_v, si_v, se_v, hist_v, tot_v):
    cid = lax.axis_index("c")
    sid = lax.axis_index("s")
    wid = sid * NCORES + cid
    lane = lax.iota(jnp.int32, L)
    ones = jnp.ones((L,), jnp.int32)
    iz = jnp.zeros((L,), jnp.int32)
    fz = jnp.zeros((L,), jnp.float32)

    def clear(i, _):
        hist_v[pl.ds(i * L, L)] = iz
        return 0

    def row_body(j, _):
        row = wid * RPW + j
        pltpu.sync_copy(in_hbm.at[row], x_v)

        lax.fori_loop(0, HWORDS // L, clear, 0)

        def p1(i, macc):
            x = x_v[pl.ds(i * L, L)]
            b = plsc.bitcast(x, jnp.int32)
            skey = b ^ ((b >> 31) & M7F)
            sk_v[pl.ds(i * L, L)] = skey
            d0 = ((skey >> 24) & 255) ^ 128
            plsc.addupdate_scatter(hist_v, [lane * 256 + d0], ones)
            return jnp.maximum(macc, x)

        macc = lax.fori_loop(
            0, NCHUNK, p1, jnp.full((L,), -jnp.inf, jnp.float32)
        )
        rank = jnp.full((L,), TOPK, jnp.int32)
        b0, rank = _bin_scan(hist_v, tot_v, rank)

        lax.fori_loop(0, HWORDS // L, clear, 0)

        def p2(i, off):
            skey = sk_v[pl.ds(i * L, L)]
            d0 = ((skey >> 24) & 255) ^ 128
            m = d0 == b0
            d1 = (skey >> 16) & 255
            plsc.addupdate_scatter(hist_v, [lane * 256 + d1], ones, mask=m)
            plsc.store_compressed(ci_v.at[pl.ds(off, L)], i * L + lane, mask=m)
            return off + jnp.max(plsc.all_reduce_population_count(m))

        n1 = lax.fori_loop(0, NCHUNK, p2, jnp.int32(0))
        b1, rank = _bin_scan(hist_v, tot_v, rank)

        lax.fori_loop(0, HWORDS // L, clear, 0)

        def p3(i, off):
            idxv = ci_v[pl.ds(i * L, L)]
            valid = (i * L + lane) < n1
            skey = plsc.load_gather(sk_v, [idxv], mask=valid)
            m = valid & (((skey >> 16) & 255) == b1)
            d2 = (skey >> 8) & 255
            plsc.addupdate_scatter(hist_v, [lane * 256 + d2], ones, mask=m)
            plsc.store_compressed(ci_v.at[pl.ds(off, L)], idxv, mask=m)
            return off + jnp.max(plsc.all_reduce_population_count(m))

        n2 = lax.fori_loop(0, (n1 + L - 1) // L, p3, jnp.int32(0))
        b2, rank = _bin_scan(hist_v, tot_v, rank)

        lax.fori_loop(0, HWORDS // L, clear, 0)

        def p4(i, _):
            idxv = ci_v[pl.ds(i * L, L)]
            valid = (i * L + lane) < n2
            skey = plsc.load_gather(sk_v, [idxv], mask=valid)
            m = valid & (((skey >> 8) & 255) == b2)
            d3 = skey & 255
            plsc.addupdate_scatter(hist_v, [lane * 256 + d3], ones, mask=m)
            return 0

        lax.fori_loop(0, (n2 + L - 1) // L, p4, 0)
        b3, budget = _bin_scan(hist_v, tot_v, rank)
        t_skey = (((b0 * 256 + b1) * 256 + b2) * 256 + b3) ^ MINT

        def p5(i, off):
            skey = sk_v[pl.ds(i * L, L)]
            m = skey > t_skey
            plsc.store_compressed(si_v.at[pl.ds(off, L)], i * L + lane, mask=m)
            return off + jnp.max(plsc.all_reduce_population_count(m))

        ngt = lax.fori_loop(0, NCHUNK, p5, jnp.int32(0))

        def p5b(i, carry):
            off, seen = carry
            idxv = ci_v[pl.ds(i * L, L)]
            valid = (i * L + lane) < n2
            skey = plsc.load_gather(sk_v, [idxv], mask=valid)
            eq = valid & (skey == t_skey)
            pos = plsc.cumsum(eq.astype(jnp.int32)) + seen
            sel = eq & (pos <= budget)
            plsc.store_compressed(si_v.at[pl.ds(off, L)], idxv, mask=sel)
            off = off + jnp.max(plsc.all_reduce_population_count(sel))
            return off, seen + plsc.all_reduce_population_count(eq)

        lax.fori_loop(0, (n2 + L - 1) // L, p5b, (ngt, iz))

        mx = jnp.max(macc)

        def pexp(t, ssum):
            xv = plsc.load_gather(x_v, [si_v[pl.ds(t * L, L)]])
            e = jnp.exp(xv - mx)
            se_v[pl.ds(t * L, L)] = e
            return ssum + jnp.sum(e)

        ssum = lax.fori_loop(0, TOPK // L, pexp, jnp.float32(0))
        inv = 1.0 / ssum

        def pz(i, _):
            x_v[pl.ds(i * L, L)] = fz
            return 0

        lax.fori_loop(0, NCHUNK, pz, 0)

        def psc(t, _):
            idxv = si_v[pl.ds(t * L, L)]
            plsc.store_scatter(x_v, [idxv], se_v[pl.ds(t * L, L)] * inv)
            return 0

        lax.fori_loop(0, TOPK // L, psc, 0)
        pltpu.sync_copy(x_v, out_hbm.at[row])
        return 0

    lax.fori_loop(0, RPW, row_body, 0)


def _make(interpret=False):
    mesh = plsc.VectorSubcoreMesh(core_axis_name="c", subcore_axis_name="s")
    return pl.kernel(
        _body,
        out_type=jax.ShapeDtypeStruct((ROWS, N), jnp.float32),
        mesh=mesh,
        scratch_types=[
            pltpu.VMEM((N,), jnp.float32),
            pltpu.VMEM((N,), jnp.int32),
            pltpu.VMEM((N + 2 * L,), jnp.int32),
            pltpu.VMEM((6 * L,), jnp.int32),
            pltpu.VMEM((TOPK,), jnp.float32),
            pltpu.VMEM((HWORDS,), jnp.int32),
            pltpu.VMEM((256,), jnp.int32),
        ],
        interpret=interpret,
    )


_pk = _make()


@jax.jit
def kernel(inputs):
    return _pk(inputs)


# speedup vs baseline: 3.6595x; 3.6595x over previous
import functools

import numpy as np
import jax
import jax.numpy as jnp
from jax import lax
from jax.experimental import pallas as pl
from jax.experimental.pallas import tpu as pltpu
from jax.experimental.pallas import tpu_sc as plsc

ROWS = 128
N = 8192
TOPK = 64
L = 16
NCHUNK = N // L
NCORES = 2
NSUB = 16
NW = NCORES * NSUB
RPW = ROWS // NW
HWORDS = L * 256

MINT = np.int32(-2147483648)
M7F = np.int32(0x7FFFFFFF)


def _bin_scan(hist_v, tot_v, rank):
    lane = lax.iota(jnp.int32, L)

    def tot_body(g, _):
        def lane_body(l, acc):
            return acc + hist_v[pl.ds(l * 256 + g * L, L)]

        tot_v[pl.ds(g * L, L)] = lax.fori_loop(
            0, L, lane_body, jnp.zeros((L,), jnp.int32)
        )
        return 0

    lax.fori_loop(0, 16, tot_body, 0)

    def sweep(g9, carry):
        carry_cnt, pc_tot = carry
        g = 15 - g9
        v = tot_v[pl.ds(g * L, L)]
        sfx = lax.rev(plsc.cumsum(lax.rev(v, (0,))), (0,))
        m = (sfx + carry_cnt) >= rank
        pc_tot = pc_tot + plsc.all_reduce_population_count(m)
        return carry_cnt + jnp.sum(v), pc_tot

    _, pc_tot = lax.fori_loop(
        0, 16, sweep, (jnp.int32(0), jnp.zeros((L,), jnp.int32))
    )
    bstar = pc_tot - 1

    def above(g, acc):
        v = tot_v[pl.ds(g * L, L)]
        binidx = g * L + lane
        return acc + jnp.sum(jnp.where(binidx > bstar, v, 0))

    cnt_above = lax.fori_loop(0, 16, above, jnp.int32(0))
    return bstar, rank - cnt_above


def _body(in_hbm, out_hbm, x_v, sk_v, ci_v, si_v, se_v, hist_v, tot_v):
    cid = lax.axis_index("c")
    sid = lax.axis_index("s")
    wid = sid * NCORES + cid
    lane = lax.iota(jnp.int32, L)
    ones = jnp.ones((L,), jnp.int32)
    iz = jnp.zeros((L,), jnp.int32)
    fz = jnp.zeros((L,), jnp.float32)

    def clear(i, _):
        hist_v[pl.ds(i * L, L)] = iz
        return 0

    def row_body(j, _):
        row = wid * RPW + j
        pltpu.sync_copy(in_hbm.at[row], x_v)

        lax.fori_loop(0, HWORDS // L, clear, 0)

        def p1(i, macc):
            x = x_v[pl.ds(i * L, L)]
            b = lax.bitcast_convert_type(x, jnp.int32)
            skey = b ^ ((b >> 31) & M7F)
            sk_v[pl.ds(i * L, L)] = skey
            d0 = ((skey >> 24) & 255) ^ 128
            plsc.addupdate_scatter(hist_v, [lane * 256 + d0], ones)
            return jnp.maximum(macc, x)

        macc = lax.fori_loop(
            0, NCHUNK, p1, jnp.full((L,), -jnp.inf, jnp.float32)
        )
        rank = jnp.full((L,), TOPK, jnp.int32)
        b0, rank = _bin_scan(hist_v, tot_v, rank)

        lax.fori_loop(0, HWORDS // L, clear, 0)

        def p2(i, off):
            skey = sk_v[pl.ds(i * L, L)]
            d0 = ((skey >> 24) & 255) ^ 128
            m = d0 == b0
            d1 = (skey >> 16) & 255
            plsc.addupdate_scatter(hist_v, [lane * 256 + d1], ones, mask=m)
            plsc.store_compressed(ci_v.at[pl.ds(off, L)], i * L + lane, mask=m)
            return off + jnp.max(plsc.all_reduce_population_count(m))

        n1 = lax.fori_loop(0, NCHUNK, p2, jnp.int32(0))
        b1, rank = _bin_scan(hist_v, tot_v, rank)

        lax.fori_loop(0, HWORDS // L, clear, 0)

        def p3(i, off):
            idxv = ci_v[pl.ds(i * L, L)]
            valid = (i * L + lane) < n1
            skey = plsc.load_gather(sk_v, [idxv], mask=valid)
            m = valid & (((skey >> 16) & 255) == b1)
            d2 = (skey >> 8) & 255
            plsc.addupdate_scatter(hist_v, [lane * 256 + d2], ones, mask=m)
            plsc.store_compressed(ci_v.at[pl.ds(off, L)], idxv, mask=m)
            return off + jnp.max(plsc.all_reduce_population_count(m))

        n2 = lax.fori_loop(0, (n1 + L - 1) // L, p3, jnp.int32(0))
        b2, rank = _bin_scan(hist_v, tot_v, rank)

        lax.fori_loop(0, HWORDS // L, clear, 0)

        def p4(i, _):
            idxv = ci_v[pl.ds(i * L, L)]
            valid = (i * L + lane) < n2
            skey = plsc.load_gather(sk_v, [idxv], mask=valid)
            m = valid & (((skey >> 8) & 255) == b2)
            d3 = skey & 255
            plsc.addupdate_scatter(hist_v, [lane * 256 + d3], ones, mask=m)
            return 0

        lax.fori_loop(0, (n2 + L - 1) // L, p4, 0)
        b3, budget = _bin_scan(hist_v, tot_v, rank)
        t_skey = (((b0 * 256 + b1) * 256 + b2) * 256 + b3) ^ MINT

        def p5(i, off):
            skey = sk_v[pl.ds(i * L, L)]
            m = skey > t_skey
            plsc.store_compressed(si_v.at[pl.ds(off, L)], i * L + lane, mask=m)
            return off + jnp.max(plsc.all_reduce_population_count(m))

        ngt = lax.fori_loop(0, NCHUNK, p5, jnp.int32(0))

        def p5b(i, carry):
            off, seen = carry
            idxv = ci_v[pl.ds(i * L, L)]
            valid = (i * L + lane) < n2
            skey = plsc.load_gather(sk_v, [idxv], mask=valid)
            eq = valid & (skey == t_skey)
            pos = plsc.cumsum(eq.astype(jnp.int32)) + seen
            sel = eq & (pos <= budget)
            plsc.store_compressed(si_v.at[pl.ds(off, L)], idxv, mask=sel)
            off = off + jnp.max(plsc.all_reduce_population_count(sel))
            return off, seen + plsc.all_reduce_population_count(eq)

        lax.fori_loop(0, (n2 + L - 1) // L, p5b, (ngt, iz))

        mx = jnp.max(macc)

        def pexp(t, ssum):
            xv = plsc.load_gather(x_v, [si_v[pl.ds(t * L, L)]])
            e = jnp.exp(xv - mx)
            se_v[pl.ds(t * L, L)] = e
            return ssum + jnp.sum(e)

        ssum = lax.fori_loop(0, TOPK // L, pexp, jnp.float32(0))
        inv = jnp.ones((L,), jnp.float32) / jnp.broadcast_to(ssum, (L,))

        def pz(i, _):
            x_v[pl.ds(i * L, L)] = fz
            return 0

        lax.fori_loop(0, NCHUNK, pz, 0)

        def psc(t, _):
            idxv = si_v[pl.ds(t * L, L)]
            plsc.store_scatter(x_v, [idxv], se_v[pl.ds(t * L, L)] * inv)
            return 0

        lax.fori_loop(0, TOPK // L, psc, 0)
        pltpu.sync_copy(x_v, out_hbm.at[row])
        return 0

    lax.fori_loop(0, RPW, row_body, 0)


def _make(interpret=False):
    mesh = plsc.VectorSubcoreMesh(
        core_axis_name="c", subcore_axis_name="s",
        num_cores=NCORES, num_subcores=NSUB,
    )
    return pl.kernel(
        _body,
        out_type=jax.ShapeDtypeStruct((ROWS, N), jnp.float32),
        mesh=mesh,
        scratch_types=[
            pltpu.VMEM((N,), jnp.float32),
            pltpu.VMEM((N,), jnp.int32),
            pltpu.VMEM((N + 2 * L,), jnp.int32),
            pltpu.VMEM((6 * L,), jnp.int32),
            pltpu.VMEM((TOPK,), jnp.float32),
            pltpu.VMEM((HWORDS,), jnp.int32),
            pltpu.VMEM((256,), jnp.int32),
        ],
        compiler_params=pltpu.CompilerParams(needs_layout_passes=False),
        interpret=interpret,
    )


_pk = _make()


@jax.jit
def kernel(inputs):
    return _pk(inputs)


# speedup vs baseline: 4.3990x; 1.2021x over previous
import functools

import numpy as np
import jax
import jax.numpy as jnp
from jax import lax
from jax.experimental import pallas as pl
from jax.experimental.pallas import tpu as pltpu
from jax.experimental.pallas import tpu_sc as plsc

ROWS = 128
N = 8192
TOPK = 64
L = 16
NCHUNK = N // L
NCORES = 2
NSUB = 16
NW = NCORES * NSUB
RPW = ROWS // NW
NBINS = 256

MINT = np.int32(-2147483648)
M7F = np.int32(0x7FFFFFFF)


def _pc0(m):
    return plsc.all_reduce_population_count(m)[0]


def _bin_scan(hist_v, rank):
    lane = lax.iota(jnp.int32, L)

    def sweep(g9, carry):
        carry_cnt, pc_tot = carry
        g = 15 - g9
        v = hist_v[pl.ds(g * L, L)]
        sfx = lax.rev(plsc.cumsum(lax.rev(v, (0,))), (0,))
        m = (sfx + carry_cnt) >= rank
        pc_tot = pc_tot + plsc.all_reduce_population_count(m)
        return carry_cnt + jnp.sum(v), pc_tot

    _, pc_tot = lax.fori_loop(
        0, 16, sweep, (jnp.int32(0), jnp.zeros((L,), jnp.int32))
    )
    bstar = pc_tot - 1

    def above(g, acc):
        v = hist_v[pl.ds(g * L, L)]
        binidx = g * L + lane
        return acc + jnp.sum(jnp.where(binidx > bstar, v, 0))

    cnt_above = lax.fori_loop(0, 16, above, jnp.int32(0))
    return bstar, rank - cnt_above


def _body(in_hbm, out_hbm, x_v, sk_v, ci_v, si_v, se_v, hist_v):
    cid = lax.axis_index("c")
    sid = lax.axis_index("s")
    wid = sid * NCORES + cid
    lane = lax.iota(jnp.int32, L)
    ones = jnp.ones((L,), jnp.int32)
    iz = jnp.zeros((L,), jnp.int32)
    fz = jnp.zeros((L,), jnp.float32)

    def clear(i, _):
        hist_v[pl.ds(i * L, L)] = iz
        return 0

    def row_body(j, _):
        row = wid * RPW + j
        pltpu.sync_copy(in_hbm.at[row], x_v)

        lax.fori_loop(0, NBINS // L, clear, 0)

        def p1(i, macc):
            x = x_v[pl.ds(i * L, L)]
            b = lax.bitcast_convert_type(x, jnp.int32)
            skey = b ^ ((b >> 31) & M7F)
            sk_v[pl.ds(i * L, L)] = skey
            d0 = ((skey >> 24) & 255) ^ 128
            plsc.addupdate_scatter(hist_v, [d0], ones)
            return jnp.maximum(macc, x)

        macc = lax.fori_loop(
            0, NCHUNK, p1, jnp.full((L,), -jnp.inf, jnp.float32)
        )
        rank = jnp.full((L,), TOPK, jnp.int32)
        b0, rank = _bin_scan(hist_v, rank)

        lax.fori_loop(0, NBINS // L, clear, 0)

        def p2(i, off):
            skey = sk_v[pl.ds(i * L, L)]
            d0 = ((skey >> 24) & 255) ^ 128
            m = d0 == b0
            d1 = (skey >> 16) & 255
            plsc.addupdate_scatter(hist_v, [d1], ones, mask=m)
            plsc.store_compressed(ci_v.at[pl.ds(off, L)], i * L + lane, mask=m)
            return off + _pc0(m)

        n1 = lax.fori_loop(0, NCHUNK, p2, jnp.int32(0))
        b1, rank = _bin_scan(hist_v, rank)

        lax.fori_loop(0, NBINS // L, clear, 0)

        def p3(i, off):
            idxv = ci_v[pl.ds(i * L, L)]
            valid = (i * L + lane) < n1
            skey = plsc.load_gather(sk_v, [idxv], mask=valid)
            m = valid & (((skey >> 16) & 255) == b1)
            d2 = (skey >> 8) & 255
            plsc.addupdate_scatter(hist_v, [d2], ones, mask=m)
            plsc.store_compressed(ci_v.at[pl.ds(off, L)], idxv, mask=m)
            return off + _pc0(m)

        n2 = lax.fori_loop(0, (n1 + L - 1) // L, p3, jnp.int32(0))
        b2, rank = _bin_scan(hist_v, rank)

        lax.fori_loop(0, NBINS // L, clear, 0)

        def p4(i, _):
            idxv = ci_v[pl.ds(i * L, L)]
            valid = (i * L + lane) < n2
            skey = plsc.load_gather(sk_v, [idxv], mask=valid)
            m = valid & (((skey >> 8) & 255) == b2)
            d3 = skey & 255
            plsc.addupdate_scatter(hist_v, [d3], ones, mask=m)
            return 0

        lax.fori_loop(0, (n2 + L - 1) // L, p4, 0)
        b3, budget = _bin_scan(hist_v, rank)
        t_skey = (((b0 * 256 + b1) * 256 + b2) * 256 + b3) ^ MINT

        def p5(i, off):
            skey = sk_v[pl.ds(i * L, L)]
            m = skey > t_skey
            plsc.store_compressed(si_v.at[pl.ds(off, L)], i * L + lane, mask=m)
            return off + _pc0(m)

        ngt = lax.fori_loop(0, NCHUNK, p5, jnp.int32(0))

        def p5b(i, carry):
            off, seen = carry
            idxv = ci_v[pl.ds(i * L, L)]
            valid = (i * L + lane) < n2
            skey = plsc.load_gather(sk_v, [idxv], mask=valid)
            eq = valid & (skey == t_skey)
            pos = plsc.cumsum(eq.astype(jnp.int32)) + seen
            sel = eq & (pos <= budget)
            plsc.store_compressed(si_v.at[pl.ds(off, L)], idxv, mask=sel)
            return off + _pc0(sel), seen + plsc.all_reduce_population_count(eq)

        lax.fori_loop(0, (n2 + L - 1) // L, p5b, (ngt, iz))

        mx = jnp.max(macc)

        def pexp(t, ssum):
            xv = plsc.load_gather(x_v, [si_v[pl.ds(t * L, L)]])
            e = jnp.exp(xv - mx)
            se_v[pl.ds(t * L, L)] = e
            return ssum + jnp.sum(e)

        ssum = lax.fori_loop(0, TOPK // L, pexp, jnp.float32(0))
        inv = jnp.ones((L,), jnp.float32) / jnp.broadcast_to(ssum, (L,))

        def pz(i, _):
            x_v[pl.ds(i * L, L)] = fz
            return 0

        lax.fori_loop(0, NCHUNK, pz, 0)

        def psc(t, _):
            idxv = si_v[pl.ds(t * L, L)]
            plsc.store_scatter(x_v, [idxv], se_v[pl.ds(t * L, L)] * inv)
            return 0

        lax.fori_loop(0, TOPK // L, psc, 0)
        pltpu.sync_copy(x_v, out_hbm.at[row])
        return 0

    lax.fori_loop(0, RPW, row_body, 0)


def _make(interpret=False):
    mesh = plsc.VectorSubcoreMesh(
        core_axis_name="c", subcore_axis_name="s",
        num_cores=NCORES, num_subcores=NSUB,
    )
    return pl.kernel(
        _body,
        out_type=jax.ShapeDtypeStruct((ROWS, N), jnp.float32),
        mesh=mesh,
        scratch_types=[
            pltpu.VMEM((N,), jnp.float32),
            pltpu.VMEM((N,), jnp.int32),
            pltpu.VMEM((N + 2 * L,), jnp.int32),
            pltpu.VMEM((6 * L,), jnp.int32),
            pltpu.VMEM((TOPK,), jnp.float32),
            pltpu.VMEM((NBINS,), jnp.int32),
        ],
        compiler_params=pltpu.CompilerParams(needs_layout_passes=False),
        interpret=interpret,
    )


_pk = _make()


@jax.jit
def kernel(inputs):
    return _pk(inputs)


# speedup vs baseline: 5.9213x; 1.3460x over previous
import functools

import numpy as np
import jax
import jax.numpy as jnp
from jax import lax
from jax.experimental import pallas as pl
from jax.experimental.pallas import tpu as pltpu
from jax.experimental.pallas import tpu_sc as plsc

ROWS = 128
N = 8192
TOPK = 64
L = 16
NCHUNK = N // L
NCORES = 2
NSUB = 16
NW = NCORES * NSUB
RPW = ROWS // NW
NBINS = 256

MINT = np.int32(-2147483648)
M7F = np.int32(0x7FFFFFFF)


def _pc0(m):
    return plsc.all_reduce_population_count(m)[0]


def _skey(x):
    b = lax.bitcast_convert_type(x, jnp.int32)
    return b ^ ((b >> 31) & M7F)


def _bin_scan(hist_v, tot_v, rank):
    def sweep(g9, carry):
        carry_cnt, pc_tot = carry
        g = 15 - g9
        v = hist_v[pl.ds(g * L, L)]
        sfx = lax.rev(plsc.cumsum(lax.rev(v, (0,))), (0,)) + carry_cnt
        tot_v[pl.ds(g * L, L)] = sfx
        m = sfx >= rank
        pc_tot = pc_tot + plsc.all_reduce_population_count(m)
        return sfx[0], pc_tot

    _, pc_tot = lax.fori_loop(
        0, 16, sweep, (jnp.int32(0), jnp.zeros((L,), jnp.int32))
    )
    bstar = pc_tot - 1
    nxt = jnp.minimum(bstar + 1, jnp.broadcast_to(jnp.int32(255), (L,)))
    cnt_above = jnp.where(
        bstar >= 255, jnp.zeros((L,), jnp.int32), plsc.load_gather(tot_v, [nxt])
    )
    return bstar, rank - cnt_above


def _body(in_hbm, out_hbm, x_v, out_v, ci_v, si_v, se_v, hist_v, tot_v):
    cid = lax.axis_index("c")
    sid = lax.axis_index("s")
    wid = sid * NCORES + cid
    lane = lax.iota(jnp.int32, L)
    ones = jnp.ones((L,), jnp.int32)
    iz = jnp.zeros((L,), jnp.int32)
    fz = jnp.zeros((L,), jnp.float32)

    def clear_hist(i, _):
        hist_v[pl.ds(i * L, L)] = iz
        return 0

    def zout(i, _):
        out_v[pl.ds(i * L, L)] = fz
        return 0

    lax.fori_loop(0, NCHUNK, zout, 0)

    def row_body(j, _):
        row = wid * RPW + j
        pltpu.sync_copy(in_hbm.at[row], x_v)

        lax.fori_loop(0, NBINS // L, clear_hist, 0)

        def p1(i, macc):
            x = x_v[pl.ds(i * L, L)]
            d0 = ((_skey(x) >> 24) & 255) ^ 128
            plsc.addupdate_scatter(hist_v, [d0], ones)
            return jnp.maximum(macc, x)

        macc = lax.fori_loop(
            0, NCHUNK, p1, jnp.full((L,), -jnp.inf, jnp.float32)
        )
        rank = jnp.full((L,), TOPK, jnp.int32)
        b0, rank = _bin_scan(hist_v, tot_v, rank)

        def p2(i, carry):
            goff, coff = carry
            x = x_v[pl.ds(i * L, L)]
            d0 = ((_skey(x) >> 24) & 255) ^ 128
            mg = d0 > b0
            me = d0 == b0
            idxv = i * L + lane
            plsc.store_compressed(si_v.at[pl.ds(goff, L)], idxv, mask=mg)
            plsc.store_compressed(ci_v.at[pl.ds(coff, L)], idxv, mask=me)
            return goff + _pc0(mg), coff + _pc0(me)

        ngt, n1 = lax.fori_loop(0, NCHUNK, p2, (jnp.int32(0), jnp.int32(0)))

        lax.fori_loop(0, NBINS // L, clear_hist, 0)

        def p2b(i, _):
            valid = (i * L + lane) < n1
            idxv = ci_v[pl.ds(i * L, L)]
            skey = _skey(plsc.load_gather(x_v, [idxv], mask=valid))
            d1 = (skey >> 16) & 255
            plsc.addupdate_scatter(hist_v, [d1], ones, mask=valid)
            return 0

        lax.fori_loop(0, (n1 + L - 1) // L, p2b, 0)
        b1, rank = _bin_scan(hist_v, tot_v, rank)

        lax.fori_loop(0, NBINS // L, clear_hist, 0)

        def p3(i, carry):
            goff, coff = carry
            valid = (i * L + lane) < n1
            idxv = ci_v[pl.ds(i * L, L)]
            skey = _skey(plsc.load_gather(x_v, [idxv], mask=valid))
            d1 = (skey >> 16) & 255
            mg = valid & (d1 > b1)
            me = valid & (d1 == b1)
            d2 = (skey >> 8) & 255
            plsc.addupdate_scatter(hist_v, [d2], ones, mask=me)
            plsc.store_compressed(si_v.at[pl.ds(goff, L)], idxv, mask=mg)
            plsc.store_compressed(ci_v.at[pl.ds(coff, L)], idxv, mask=me)
            return goff + _pc0(mg), coff + _pc0(me)

        ngt, n2 = lax.fori_loop(
            0, (n1 + L - 1) // L, p3, (ngt, jnp.int32(0))
        )
        b2, rank = _bin_scan(hist_v, tot_v, rank)

        lax.fori_loop(0, NBINS // L, clear_hist, 0)

        def p4(i, carry):
            goff, coff = carry
            valid = (i * L + lane) < n2
            idxv = ci_v[pl.ds(i * L, L)]
            skey = _skey(plsc.load_gather(x_v, [idxv], mask=valid))
            d2 = (skey >> 8) & 255
            mg = valid & (d2 > b2)
            me = valid & (d2 == b2)
            d3 = skey & 255
            plsc.addupdate_scatter(hist_v, [d3], ones, mask=me)
            plsc.store_compressed(si_v.at[pl.ds(goff, L)], idxv, mask=mg)
            plsc.store_compressed(ci_v.at[pl.ds(coff, L)], idxv, mask=me)
            return goff + _pc0(mg), coff + _pc0(me)

        ngt, n3 = lax.fori_loop(
            0, (n2 + L - 1) // L, p4, (ngt, jnp.int32(0))
        )
        b3, budget = _bin_scan(hist_v, tot_v, rank)

        def p5(i, carry):
            goff, seen = carry
            valid = (i * L + lane) < n3
            idxv = ci_v[pl.ds(i * L, L)]
            skey = _skey(plsc.load_gather(x_v, [idxv], mask=valid))
            d3 = skey & 255
            mg = valid & (d3 > b3)
            plsc.store_compressed(si_v.at[pl.ds(goff, L)], idxv, mask=mg)
            goff = goff + _pc0(mg)
            eq = valid & (d3 == b3)
            pos = plsc.cumsum(eq.astype(jnp.int32)) + seen
            sel = eq & (pos <= budget)
            plsc.store_compressed(si_v.at[pl.ds(goff, L)], idxv, mask=sel)
            return goff + _pc0(sel), seen + plsc.all_reduce_population_count(eq)

        lax.fori_loop(0, (n3 + L - 1) // L, p5, (ngt, iz))

        mx = jnp.max(macc)

        def pexp(t, ssum):
            xv = plsc.load_gather(x_v, [si_v[pl.ds(t * L, L)]])
            e = jnp.exp(xv - mx)
            se_v[pl.ds(t * L, L)] = e
            return ssum + jnp.sum(e)

        ssum = lax.fori_loop(0, TOPK // L, pexp, jnp.float32(0))
        inv = jnp.ones((L,), jnp.float32) / jnp.broadcast_to(ssum, (L,))

        def psc(t, _):
            idxv = si_v[pl.ds(t * L, L)]
            plsc.store_scatter(out_v, [idxv], se_v[pl.ds(t * L, L)] * inv)
            return 0

        lax.fori_loop(0, TOPK // L, psc, 0)
        pltpu.sync_copy(out_v, out_hbm.at[row])

        def pzero(t, _):
            plsc.store_scatter(out_v, [si_v[pl.ds(t * L, L)]], fz)
            return 0

        lax.fori_loop(0, TOPK // L, pzero, 0)
        return 0

    lax.fori_loop(0, RPW, row_body, 0)


def _make(interpret=False):
    mesh = plsc.VectorSubcoreMesh(
        core_axis_name="c", subcore_axis_name="s",
        num_cores=NCORES, num_subcores=NSUB,
    )
    return pl.kernel(
        _body,
        out_type=jax.ShapeDtypeStruct((ROWS, N), jnp.float32),
        mesh=mesh,
        scratch_types=[
            pltpu.VMEM((N,), jnp.float32),
            pltpu.VMEM((N,), jnp.float32),
            pltpu.VMEM((N + 2 * L,), jnp.int32),
            pltpu.VMEM((6 * L,), jnp.int32),
            pltpu.VMEM((TOPK,), jnp.float32),
            pltpu.VMEM((NBINS,), jnp.int32),
            pltpu.VMEM((NBINS,), jnp.int32),
        ],
        compiler_params=pltpu.CompilerParams(needs_layout_passes=False),
        interpret=interpret,
    )


_pk = _make()


@jax.jit
def kernel(inputs):
    return _pk(inputs)


# speedup vs baseline: 8.1019x; 1.3683x over previous
import functools

import numpy as np
import jax
import jax.numpy as jnp
from jax import lax
from jax.experimental import pallas as pl
from jax.experimental.pallas import tpu as pltpu
from jax.experimental.pallas import tpu_sc as plsc

ROWS = 128
N = 8192
TOPK = 64
L = 16
NCHUNK = N // L
UNROLL = 4
NCORES = 2
NSUB = 16
NW = NCORES * NSUB
RPW = ROWS // NW
NBINS = 256

MINT = np.int32(-2147483648)
M7F = np.int32(0x7FFFFFFF)


def _pc0(m):
    return plsc.all_reduce_population_count(m)[0]


def _skey(x):
    b = lax.bitcast_convert_type(x, jnp.int32)
    return b ^ ((b >> 31) & M7F)


def _bin_scan(hist_v, tot_v, rank):
    def sweep(g9, carry):
        carry_cnt, pc_tot = carry
        g = 15 - g9
        v = hist_v[pl.ds(g * L, L)]
        sfx = lax.rev(plsc.cumsum(lax.rev(v, (0,))), (0,)) + carry_cnt
        tot_v[pl.ds(g * L, L)] = sfx
        m = sfx >= rank
        pc_tot = pc_tot + plsc.all_reduce_population_count(m)
        return sfx[0], pc_tot

    _, pc_tot = lax.fori_loop(
        0, 16, sweep, (jnp.int32(0), jnp.zeros((L,), jnp.int32))
    )
    bstar = pc_tot - 1
    nxt = jnp.minimum(bstar + 1, jnp.broadcast_to(jnp.int32(255), (L,)))
    cnt_above = jnp.where(
        bstar >= 255, jnp.zeros((L,), jnp.int32), plsc.load_gather(tot_v, [nxt])
    )
    return bstar, rank - cnt_above


def _body(in_hbm, out_hbm, x_v, out_v, ci_v, si_v, se_v, hist_v, tot_v):
    cid = lax.axis_index("c")
    sid = lax.axis_index("s")
    wid = sid * NCORES + cid
    lane = lax.iota(jnp.int32, L)
    ones = jnp.ones((L,), jnp.int32)
    iz = jnp.zeros((L,), jnp.int32)
    fz = jnp.zeros((L,), jnp.float32)

    def clear_hist(i, _):
        hist_v[pl.ds(i * L, L)] = iz
        return 0

    def zout(i, _):
        out_v[pl.ds(i * L, L)] = fz
        return 0

    lax.fori_loop(0, NCHUNK, zout, 0)

    def row_body(j, _):
        row = wid * RPW + j
        pltpu.sync_copy(in_hbm.at[row], x_v)

        def pa(i, accs):
            base = i * (L * UNROLL)
            return tuple(
                jnp.maximum(accs[k], x_v[pl.ds(base + k * L, L)])
                for k in range(UNROLL)
            )

        ninf = jnp.full((L,), -jnp.inf, jnp.float32)
        accs = lax.fori_loop(
            0, NCHUNK // UNROLL, pa, (ninf,) * UNROLL
        )
        m01 = jnp.maximum(accs[0], accs[1])
        m23 = jnp.maximum(accs[2], accs[3])
        mx = jnp.max(jnp.maximum(m01, m23))
        tlb = jnp.min(
            jnp.minimum(
                jnp.minimum(accs[0], accs[1]), jnp.minimum(accs[2], accs[3])
            )
        )
        tlb_s = jnp.broadcast_to(tlb, (L,))

        def pb(i, off):
            base = i * (L * UNROLL)
            for k in range(UNROLL):
                x = x_v[pl.ds(base + k * L, L)]
                m = x >= tlb_s
                plsc.store_compressed(
                    ci_v.at[pl.ds(off, L)], base + k * L + lane, mask=m
                )
                off = off + _pc0(m)
            return off

        n0 = lax.fori_loop(0, NCHUNK // UNROLL, pb, jnp.int32(0))

        lax.fori_loop(0, NBINS // L, clear_hist, 0)

        def ph0(i, _):
            valid = (i * L + lane) < n0
            idxv = ci_v[pl.ds(i * L, L)]
            skey = _skey(plsc.load_gather(x_v, [idxv], mask=valid))
            d0 = ((skey >> 24) & 255) ^ 128
            plsc.addupdate_scatter(hist_v, [d0], ones, mask=valid)
            return 0

        lax.fori_loop(0, (n0 + L - 1) // L, ph0, 0)
        rank = jnp.full((L,), TOPK, jnp.int32)
        b0, rank = _bin_scan(hist_v, tot_v, rank)

        lax.fori_loop(0, NBINS // L, clear_hist, 0)

        def p30(i, carry):
            goff, coff = carry
            valid = (i * L + lane) < n0
            idxv = ci_v[pl.ds(i * L, L)]
            skey = _skey(plsc.load_gather(x_v, [idxv], mask=valid))
            d0 = ((skey >> 24) & 255) ^ 128
            mg = valid & (d0 > b0)
            me = valid & (d0 == b0)
            d1 = (skey >> 16) & 255
            plsc.addupdate_scatter(hist_v, [d1], ones, mask=me)
            plsc.store_compressed(si_v.at[pl.ds(goff, L)], idxv, mask=mg)
            plsc.store_compressed(ci_v.at[pl.ds(coff, L)], idxv, mask=me)
            return goff + _pc0(mg), coff + _pc0(me)

        ngt, n1 = lax.fori_loop(
            0, (n0 + L - 1) // L, p30, (jnp.int32(0), jnp.int32(0))
        )
        b1, rank = _bin_scan(hist_v, tot_v, rank)

        lax.fori_loop(0, NBINS // L, clear_hist, 0)

        def p31(i, carry):
            goff, coff = carry
            valid = (i * L + lane) < n1
            idxv = ci_v[pl.ds(i * L, L)]
            skey = _skey(plsc.load_gather(x_v, [idxv], mask=valid))
            d1 = (skey >> 16) & 255
            mg = valid & (d1 > b1)
            me = valid & (d1 == b1)
            d2 = (skey >> 8) & 255
            plsc.addupdate_scatter(hist_v, [d2], ones, mask=me)
            plsc.store_compressed(si_v.at[pl.ds(goff, L)], idxv, mask=mg)
            plsc.store_compressed(ci_v.at[pl.ds(coff, L)], idxv, mask=me)
            return goff + _pc0(mg), coff + _pc0(me)

        ngt, n2 = lax.fori_loop(
            0, (n1 + L - 1) // L, p31, (ngt, jnp.int32(0))
        )
        b2, rank = _bin_scan(hist_v, tot_v, rank)

        lax.fori_loop(0, NBINS // L, clear_hist, 0)

        def p32(i, carry):
            goff, coff = carry
            valid = (i * L + lane) < n2
            idxv = ci_v[pl.ds(i * L, L)]
            skey = _skey(plsc.load_gather(x_v, [idxv], mask=valid))
            d2 = (skey >> 8) & 255
            mg = valid & (d2 > b2)
            me = valid & (d2 == b2)
            d3 = skey & 255
            plsc.addupdate_scatter(hist_v, [d3], ones, mask=me)
            plsc.store_compressed(si_v.at[pl.ds(goff, L)], idxv, mask=mg)
            plsc.store_compressed(ci_v.at[pl.ds(coff, L)], idxv, mask=me)
            return goff + _pc0(mg), coff + _pc0(me)

        ngt, n3 = lax.fori_loop(
            0, (n2 + L - 1) // L, p32, (ngt, jnp.int32(0))
        )
        b3, budget = _bin_scan(hist_v, tot_v, rank)

        def p5(i, carry):
            goff, seen = carry
            valid = (i * L + lane) < n3
            idxv = ci_v[pl.ds(i * L, L)]
            skey = _skey(plsc.load_gather(x_v, [idxv], mask=valid))
            d3 = skey & 255
            mg = valid & (d3 > b3)
            plsc.store_compressed(si_v.at[pl.ds(goff, L)], idxv, mask=mg)
            goff = goff + _pc0(mg)
            eq = valid & (d3 == b3)
            pos = plsc.cumsum(eq.astype(jnp.int32)) + seen
            sel = eq & (pos <= budget)
            plsc.store_compressed(si_v.at[pl.ds(goff, L)], idxv, mask=sel)
            return goff + _pc0(sel), seen + plsc.all_reduce_population_count(eq)

        lax.fori_loop(0, (n3 + L - 1) // L, p5, (ngt, iz))

        def pexp(t, ssum):
            xv = plsc.load_gather(x_v, [si_v[pl.ds(t * L, L)]])
            e = jnp.exp(xv - mx)
            se_v[pl.ds(t * L, L)] = e
            return ssum + jnp.sum(e)

        ssum = lax.fori_loop(0, TOPK // L, pexp, jnp.float32(0))
        inv = jnp.ones((L,), jnp.float32) / jnp.broadcast_to(ssum, (L,))

        def psc(t, _):
            idxv = si_v[pl.ds(t * L, L)]
            plsc.store_scatter(out_v, [idxv], se_v[pl.ds(t * L, L)] * inv)
            return 0

        lax.fori_loop(0, TOPK // L, psc, 0)
        pltpu.sync_copy(out_v, out_hbm.at[row])

        def pzero(t, _):
            plsc.store_scatter(out_v, [si_v[pl.ds(t * L, L)]], fz)
            return 0

        lax.fori_loop(0, TOPK // L, pzero, 0)
        return 0

    lax.fori_loop(0, RPW, row_body, 0)


def _make(interpret=False):
    mesh = plsc.VectorSubcoreMesh(
        core_axis_name="c", subcore_axis_name="s",
        num_cores=NCORES, num_subcores=NSUB,
    )
    return pl.kernel(
        _body,
        out_type=jax.ShapeDtypeStruct((ROWS, N), jnp.float32),
        mesh=mesh,
        scratch_types=[
            pltpu.VMEM((N,), jnp.float32),
            pltpu.VMEM((N,), jnp.float32),
            pltpu.VMEM((N + 2 * L,), jnp.int32),
            pltpu.VMEM((6 * L,), jnp.int32),
            pltpu.VMEM((TOPK,), jnp.float32),
            pltpu.VMEM((NBINS,), jnp.int32),
            pltpu.VMEM((NBINS,), jnp.int32),
        ],
        compiler_params=pltpu.CompilerParams(needs_layout_passes=False),
        interpret=interpret,
    )


_pk = _make()


@jax.jit
def kernel(inputs):
    return _pk(inputs)


# speedup vs baseline: 8.1786x; 1.0095x over previous
import functools

import numpy as np
import jax
import jax.numpy as jnp
from jax import lax
from jax.experimental import pallas as pl
from jax.experimental.pallas import tpu as pltpu
from jax.experimental.pallas import tpu_sc as plsc

ROWS = 128
N = 8192
TOPK = 64
L = 16
NCHUNK = N // L
UNROLL = 8
NCORES = 2
NSUB = 16
NW = NCORES * NSUB
RPW = ROWS // NW
NBINS = 256

MINT = np.int32(-2147483648)
M7F = np.int32(0x7FFFFFFF)


def _pc0(m):
    return plsc.all_reduce_population_count(m)[0]


def _skey(x):
    b = lax.bitcast_convert_type(x, jnp.int32)
    return b ^ ((b >> 31) & M7F)


def _bin_scan(hist_v, gs_v, rank):
    lane = lax.iota(jnp.int32, L)

    lane0 = lane < 1

    def gsum(g, _):
        s = jnp.broadcast_to(jnp.sum(hist_v[pl.ds(g * L, L)]), (L,))
        plsc.store_scatter(gs_v, [jnp.broadcast_to(g, (L,))], s, mask=lane0)
        return 0

    lax.fori_loop(0, 16, gsum, 0)
    gs = gs_v[pl.ds(0, L)]
    sfxg = lax.rev(plsc.cumsum(lax.rev(gs, (0,))), (0,))
    gstar = plsc.all_reduce_population_count(sfxg >= rank) - 1
    carry = jnp.sum(jnp.where(lane > gstar, gs, 0))
    v = hist_v[pl.ds(gstar[0] * L, L)]
    sfx = lax.rev(plsc.cumsum(lax.rev(v, (0,))), (0,)) + carry
    p = plsc.all_reduce_population_count(sfx >= rank) - 1
    bstar = gstar * L + p
    cnt_above = jnp.sum(jnp.where(lane > p, v, 0)) + carry
    return bstar, rank - cnt_above


def _body(in_hbm, out_hbm, x_v, out_v, ci_v, si_v, se_v, hist_v, gs_v):
    cid = lax.axis_index("c")
    sid = lax.axis_index("s")
    wid = sid * NCORES + cid
    lane = lax.iota(jnp.int32, L)
    ones = jnp.ones((L,), jnp.int32)
    iz = jnp.zeros((L,), jnp.int32)
    fz = jnp.zeros((L,), jnp.float32)

    def clear_hist(i, _):
        hist_v[pl.ds(i * L, L)] = iz
        return 0

    def zout(i, _):
        out_v[pl.ds(i * L, L)] = fz
        return 0

    lax.fori_loop(0, NCHUNK, zout, 0)

    def row_body(j, _):
        row = wid * RPW + j
        pltpu.sync_copy(in_hbm.at[row], x_v)

        def pa(i, accs):
            base = i * (L * UNROLL)
            return tuple(
                jnp.maximum(accs[k], x_v[pl.ds(base + k * L, L)])
                for k in range(UNROLL)
            )

        ninf = jnp.full((L,), -jnp.inf, jnp.float32)
        accs = lax.fori_loop(
            0, NCHUNK // UNROLL, pa, (ninf,) * UNROLL
        )
        g4 = [jnp.maximum(accs[k], accs[k + 4]) for k in range(4)]
        m01 = jnp.maximum(g4[0], g4[1])
        m23 = jnp.maximum(g4[2], g4[3])
        mx = jnp.max(jnp.maximum(m01, m23))
        tlb = jnp.min(
            jnp.minimum(jnp.minimum(g4[0], g4[1]), jnp.minimum(g4[2], g4[3]))
        )
        tlb_s = jnp.broadcast_to(tlb, (L,))

        def pb(i, off):
            base = i * (L * UNROLL)
            for k in range(UNROLL):
                x = x_v[pl.ds(base + k * L, L)]
                m = x >= tlb_s
                plsc.store_compressed(
                    ci_v.at[pl.ds(off, L)], base + k * L + lane, mask=m
                )
                off = off + _pc0(m)
            return off

        n0 = lax.fori_loop(0, NCHUNK // UNROLL, pb, jnp.int32(0))

        lax.fori_loop(0, NBINS // L, clear_hist, 0)

        def ph0(i, _):
            valid = (i * L + lane) < n0
            idxv = ci_v[pl.ds(i * L, L)]
            skey = _skey(plsc.load_gather(x_v, [idxv], mask=valid))
            d0 = ((skey >> 24) & 255) ^ 128
            plsc.addupdate_scatter(hist_v, [d0], ones, mask=valid)
            return 0

        lax.fori_loop(0, (n0 + L - 1) // L, ph0, 0)
        rank = jnp.full((L,), TOPK, jnp.int32)
        b0, rank = _bin_scan(hist_v, gs_v, rank)

        lax.fori_loop(0, NBINS // L, clear_hist, 0)

        def p30(i, carry):
            goff, coff = carry
            valid = (i * L + lane) < n0
            idxv = ci_v[pl.ds(i * L, L)]
            skey = _skey(plsc.load_gather(x_v, [idxv], mask=valid))
            d0 = ((skey >> 24) & 255) ^ 128
            mg = valid & (d0 > b0)
            me = valid & (d0 == b0)
            d1 = (skey >> 16) & 255
            plsc.addupdate_scatter(hist_v, [d1], ones, mask=me)
            plsc.store_compressed(si_v.at[pl.ds(goff, L)], idxv, mask=mg)
            plsc.store_compressed(ci_v.at[pl.ds(coff, L)], idxv, mask=me)
            return goff + _pc0(mg), coff + _pc0(me)

        ngt, n1 = lax.fori_loop(
            0, (n0 + L - 1) // L, p30, (jnp.int32(0), jnp.int32(0))
        )
        b1, rank = _bin_scan(hist_v, gs_v, rank)

        lax.fori_loop(0, NBINS // L, clear_hist, 0)

        def p31(i, carry):
            goff, coff = carry
            valid = (i * L + lane) < n1
            idxv = ci_v[pl.ds(i * L, L)]
            skey = _skey(plsc.load_gather(x_v, [idxv], mask=valid))
            d1 = (skey >> 16) & 255
            mg = valid & (d1 > b1)
            me = valid & (d1 == b1)
            d2 = (skey >> 8) & 255
            plsc.addupdate_scatter(hist_v, [d2], ones, mask=me)
            plsc.store_compressed(si_v.at[pl.ds(goff, L)], idxv, mask=mg)
            plsc.store_compressed(ci_v.at[pl.ds(coff, L)], idxv, mask=me)
            return goff + _pc0(mg), coff + _pc0(me)

        ngt, n2 = lax.fori_loop(
            0, (n1 + L - 1) // L, p31, (ngt, jnp.int32(0))
        )
        b2, rank = _bin_scan(hist_v, gs_v, rank)

        lax.fori_loop(0, NBINS // L, clear_hist, 0)

        def p32(i, carry):
            goff, coff = carry
            valid = (i * L + lane) < n2
            idxv = ci_v[pl.ds(i * L, L)]
            skey = _skey(plsc.load_gather(x_v, [idxv], mask=valid))
            d2 = (skey >> 8) & 255
            mg = valid & (d2 > b2)
            me = valid & (d2 == b2)
            d3 = skey & 255
            plsc.addupdate_scatter(hist_v, [d3], ones, mask=me)
            plsc.store_compressed(si_v.at[pl.ds(goff, L)], idxv, mask=mg)
            plsc.store_compressed(ci_v.at[pl.ds(coff, L)], idxv, mask=me)
            return goff + _pc0(mg), coff + _pc0(me)

        ngt, n3 = lax.fori_loop(
            0, (n2 + L - 1) // L, p32, (ngt, jnp.int32(0))
        )
        b3, budget = _bin_scan(hist_v, gs_v, rank)

        def p5(i, carry):
            goff, seen = carry
            valid = (i * L + lane) < n3
            idxv = ci_v[pl.ds(i * L, L)]
            skey = _skey(plsc.load_gather(x_v, [idxv], mask=valid))
            d3 = skey & 255
            mg = valid & (d3 > b3)
            plsc.store_compressed(si_v.at[pl.ds(goff, L)], idxv, mask=mg)
            goff = goff + _pc0(mg)
            eq = valid & (d3 == b3)
            pos = plsc.cumsum(eq.astype(jnp.int32)) + seen
            sel = eq & (pos <= budget)
            plsc.store_compressed(si_v.at[pl.ds(goff, L)], idxv, mask=sel)
            return goff + _pc0(sel), seen + plsc.all_reduce_population_count(eq)

        lax.fori_loop(0, (n3 + L - 1) // L, p5, (ngt, iz))

        def pexp(t, ssum):
            xv = plsc.load_gather(x_v, [si_v[pl.ds(t * L, L)]])
            e = jnp.exp(xv - mx)
            se_v[pl.ds(t * L, L)] = e
            return ssum + jnp.sum(e)

        ssum = lax.fori_loop(0, TOPK // L, pexp, jnp.float32(0))
        inv = jnp.ones((L,), jnp.float32) / jnp.broadcast_to(ssum, (L,))

        def psc(t, _):
            idxv = si_v[pl.ds(t * L, L)]
            plsc.store_scatter(out_v, [idxv], se_v[pl.ds(t * L, L)] * inv)
            return 0

        lax.fori_loop(0, TOPK // L, psc, 0)
        pltpu.sync_copy(out_v, out_hbm.at[row])

        def pzero(t, _):
            plsc.store_scatter(out_v, [si_v[pl.ds(t * L, L)]], fz)
            return 0

        lax.fori_loop(0, TOPK // L, pzero, 0)
        return 0

    lax.fori_loop(0, RPW, row_body, 0)


def _make(interpret=False):
    mesh = plsc.VectorSubcoreMesh(
        core_axis_name="c", subcore_axis_name="s",
        num_cores=NCORES, num_subcores=NSUB,
    )
    return pl.kernel(
        _body,
        out_type=jax.ShapeDtypeStruct((ROWS, N), jnp.float32),
        mesh=mesh,
        scratch_types=[
            pltpu.VMEM((N,), jnp.float32),
            pltpu.VMEM((N,), jnp.float32),
            pltpu.VMEM((N + 2 * L,), jnp.int32),
            pltpu.VMEM((6 * L,), jnp.int32),
            pltpu.VMEM((TOPK,), jnp.float32),
            pltpu.VMEM((NBINS,), jnp.int32),
            pltpu.VMEM((L,), jnp.int32),
        ],
        compiler_params=pltpu.CompilerParams(needs_layout_passes=False),
        interpret=interpret,
    )


_pk = _make()


@jax.jit
def kernel(inputs):
    return _pk(inputs)
